# baseline (speedup 1.0000x reference)
import jax
import jax.numpy as jnp
from jax import lax
from jax.experimental import pallas as pl
from jax.experimental.pallas import tpu as pltpu
from jax.experimental.pallas import tpu_sc as plsc

_CH = 16
_HALF = 128
_NCHUNK = _HALF // _CH


def _qpool_body(rho_hbm, out_hbm,
                rbuf0, rbuf1, cdbuf0, cdbuf1, obuf0, obuf1,
                semr0, semr1, semc0, semc1, semo0, semo1):
    cid = lax.axis_index("c")
    sid = lax.axis_index("s")
    lanes = lax.iota(jnp.int32, 16)

    ins = [(rbuf0, cdbuf0, semr0, semc0), (rbuf1, cdbuf1, semr1, semc1)]
    outs = [(obuf0, semo0), (obuf1, semo1)]

    def issue(k):
        rbuf, cdbuf, semr, semc = ins[k % 2]
        i0 = 8 * cid + k
        r0 = 64 * i0
        return (
            pltpu.async_copy(rho_hbm.at[sid, pl.ds(r0, 32)], rbuf, semr),
            pltpu.async_copy(
                rho_hbm.at[sid, pl.ds(r0 + 32, 32), pl.ds(r0 + 32, 32)],
                cdbuf, semc),
        )

    pend_in = {0: issue(0)}
    pend_out = {}
    for k in range(_NCHUNK):
        if k + 1 < _NCHUNK:
            pend_in[k + 1] = issue(k + 1)
        for c in pend_in.pop(k):
            c.wait()
        rbuf, cdbuf, _, _ = ins[k % 2]
        obuf, semo = outs[k % 2]
        if k >= 2:
            pend_out.pop(k - 2).wait()
        i0 = 8 * cid + k
        base16 = 16 * i0

        for t in range(_CH):
            te = jnp.full((16,), 2 * t, jnp.int32)
            to = jnp.full((16,), 2 * t + 1, jnp.int32)

            for lp in range(16):
                av = plsc.load_gather(rbuf, [te, 64 * lp + 2 * lanes])
                obuf[t, pl.ds(16 * lp, 16)] = av

            bv = plsc.load_gather(rbuf, [to, 64 * lanes + 2 * t + 1])
            plsc.addupdate_scatter(
                obuf, [jnp.full((16,), t, jnp.int32), 16 * lanes + t], bv)

            cv = plsc.load_gather(cdbuf, [te, 2 * lanes])
            dv = plsc.load_gather(cdbuf, [to, jnp.full((16,), 2 * t + 1,
                                                       jnp.int32)])
            cd = cv + jnp.where(lanes == t, dv, jnp.float32(0))
            plsc.addupdate_scatter(
                obuf, [jnp.full((16,), t, jnp.int32), base16 + lanes], cd)

        pend_out[k] = pltpu.async_copy(
            obuf, out_hbm.at[sid, pl.ds(base16, _CH)], semo)
    pend_out.pop(_NCHUNK - 2).wait()
    pend_out.pop(_NCHUNK - 1).wait()


def kernel(rho, mask_x, mask_y, new_x, new_y):
    b = rho.shape[0]

    f = pl.kernel(
        _qpool_body,
        out_type=jax.ShapeDtypeStruct((b, 256, 256), jnp.float32),
        mesh=plsc.VectorSubcoreMesh(core_axis_name="c", subcore_axis_name="s"),
        scratch_types=(
            [pltpu.VMEM((32, 1024), jnp.float32)] * 2
            + [pltpu.VMEM((32, 32), jnp.float32)] * 2
            + [pltpu.VMEM((_CH, 256), jnp.float32)] * 2
            + [pltpu.SemaphoreType.DMA] * 6
        ),
        compiler_params=pltpu.CompilerParams(use_tc_tiling_on_sc=False,
                                             needs_layout_passes=False),
    )
    return f(rho)


# speedup vs baseline: 2.1341x; 2.1341x over previous
import jax
import jax.numpy as jnp
from jax import lax
from jax.experimental import pallas as pl
from jax.experimental.pallas import tpu as pltpu
from jax.experimental.pallas import tpu_sc as plsc

_CH = 16
_HALF = 128
_NCHUNK = _HALF // _CH


def _qpool_body(rho_hbm, out_hbm,
                rbuf0, rbuf1, cdbuf0, cdbuf1, obuf0, obuf1,
                semr0, semr1, semc0, semc1, semo0, semo1):
    cid = lax.axis_index("c")
    sid = lax.axis_index("s")
    lanes = lax.iota(jnp.int32, 16)

    ins = [(rbuf0, cdbuf0, semr0, semc0), (rbuf1, cdbuf1, semr1, semc1)]
    outs = [(obuf0, semo0), (obuf1, semo1)]

    def issue(k):
        rbuf, cdbuf, semr, semc = ins[k % 2]
        i0 = 8 * cid + k
        r0 = 64 * i0
        return (
            pltpu.async_copy(rho_hbm.at[sid, pl.ds(r0, 32)], rbuf, semr),
            pltpu.async_copy(
                rho_hbm.at[sid, pl.ds(r0 + 32, 32),
                           pl.ds((r0 + 32) // 128 * 128, 128)],
                cdbuf, semc),
        )

    pend_in = {0: issue(0)}
    pend_out = {}
    for k in range(_NCHUNK):
        if k + 1 < _NCHUNK:
            pend_in[k + 1] = issue(k + 1)
        for c in pend_in.pop(k):
            c.wait()
        rbuf, cdbuf, _, _ = ins[k % 2]
        obuf, semo = outs[k % 2]
        if k >= 2:
            pend_out.pop(k - 2).wait()
        i0 = 8 * cid + k
        base16 = 16 * i0

        for t in range(_CH):
            te = jnp.full((16,), 2 * t, jnp.int32)
            to = jnp.full((16,), 2 * t + 1, jnp.int32)

            for lp in range(16):
                av = plsc.load_gather(rbuf, [te, 64 * lp + 2 * lanes])
                obuf[t, pl.ds(16 * lp, 16)] = av

            bv = plsc.load_gather(rbuf, [to, 64 * lanes + 2 * t + 1])
            plsc.addupdate_scatter(
                obuf, [jnp.full((16,), t, jnp.int32), 16 * lanes + t], bv)

            off = (64 * i0 + 32) % 128
            cv = plsc.load_gather(cdbuf, [te, off + 2 * lanes])
            dv = plsc.load_gather(cdbuf, [to, jnp.full((16,), 2 * t + 1,
                                                       jnp.int32) + off])
            cd = cv + jnp.where(lanes == t, dv, jnp.float32(0))
            plsc.addupdate_scatter(
                obuf, [jnp.full((16,), t, jnp.int32), base16 + lanes], cd)

        pend_out[k] = pltpu.async_copy(
            obuf, out_hbm.at[sid, pl.ds(base16, _CH)], semo)
    pend_out.pop(_NCHUNK - 2).wait()
    pend_out.pop(_NCHUNK - 1).wait()


def kernel(rho, mask_x, mask_y, new_x, new_y):
    b = rho.shape[0]

    f = pl.kernel(
        _qpool_body,
        out_type=jax.ShapeDtypeStruct((b, 256, 256), jnp.float32),
        mesh=plsc.VectorSubcoreMesh(core_axis_name="c", subcore_axis_name="s"),
        scratch_types=(
            [pltpu.VMEM((32, 1024), jnp.float32)] * 2
            + [pltpu.VMEM((32, 128), jnp.float32)] * 2
            + [pltpu.VMEM((_CH, 256), jnp.float32)] * 2
            + [pltpu.SemaphoreType.DMA] * 6
        ),
        compiler_params=pltpu.CompilerParams(use_tc_tiling_on_sc=True,
                                             needs_layout_passes=False),
    )
    return f(rho)


# speedup vs baseline: 2.4280x; 1.1377x over previous
import jax
import jax.numpy as jnp
from jax import lax
from jax.experimental import pallas as pl
from jax.experimental.pallas import tpu as pltpu
from jax.experimental.pallas import tpu_sc as plsc

_CH = 16
_HALF = 128
_NCHUNK = _HALF // _CH


def _qpool_body(rho_hbm, out_hbm,
                rbuf0, rbuf1, cdbuf0, cdbuf1, obuf0, obuf1,
                semr0, semr1, semc0, semc1):
    cid = lax.axis_index("c")
    sid = lax.axis_index("s")
    lanes = lax.iota(jnp.int32, 16)

    ins = [(rbuf0, cdbuf0, semr0, semc0), (rbuf1, cdbuf1, semr1, semc1)]
    obufs = [obuf0, obuf1]

    def issue(k, p):
        rbuf, cdbuf, semr, semc = ins[p]
        i0 = 8 * cid + k
        r0 = 64 * i0
        pltpu.async_copy(rho_hbm.at[sid, pl.ds(r0, 32)], rbuf, semr)
        pltpu.async_copy(
            rho_hbm.at[sid, pl.ds(r0 + 32, 32),
                       pl.ds((r0 + 32) // 128 * 128, 128)],
            cdbuf, semc)

    def wait_in(p):
        rbuf, cdbuf, semr, semc = ins[p]
        pltpu.make_async_copy(rho_hbm.at[sid, pl.ds(0, 32)],
                              rbuf, semr).wait()
        pltpu.make_async_copy(rho_hbm.at[sid, pl.ds(0, 32), pl.ds(0, 128)],
                              cdbuf, semc).wait()

    def compute(k, p):
        rbuf, cdbuf, _, _ = ins[p]
        obuf = obufs[p]
        i0 = 8 * cid + k
        base16 = 16 * i0
        off = (64 * i0 + 32) % 128

        for t in range(_CH):
            te = jnp.full((16,), 2 * t, jnp.int32)
            to = jnp.full((16,), 2 * t + 1, jnp.int32)

            for lp in range(16):
                av = plsc.load_gather(rbuf, [te, 64 * lp + 2 * lanes])
                obuf[t, pl.ds(16 * lp, 16)] = av

            bv = plsc.load_gather(rbuf, [to, 64 * lanes + 2 * t + 1])
            plsc.addupdate_scatter(
                obuf, [jnp.full((16,), t, jnp.int32), 16 * lanes + t], bv)

            cv = plsc.load_gather(cdbuf, [te, off + 2 * lanes])
            dv = plsc.load_gather(cdbuf, [to, jnp.full((16,), 2 * t + 1,
                                                       jnp.int32) + off])
            cd = cv + jnp.where(lanes == t, dv, jnp.float32(0))
            plsc.addupdate_scatter(
                obuf, [jnp.full((16,), t, jnp.int32), base16 + lanes], cd)

        pltpu.sync_copy(obuf, out_hbm.at[sid, pl.ds(base16, _CH)])

    issue(0, 0)
    issue(1, 1)

    def pair_body(kk, carry):
        for p in range(2):
            k = 2 * kk + p
            wait_in(p)
            compute(k, p)

            @pl.when(kk < (_NCHUNK // 2) - 1)
            def _():
                issue(k + 2, p)
        return carry
    lax.fori_loop(0, _NCHUNK // 2, pair_body, 0)


def kernel(rho, mask_x, mask_y, new_x, new_y):
    b = rho.shape[0]

    f = pl.kernel(
        _qpool_body,
        out_type=jax.ShapeDtypeStruct((b, 256, 256), jnp.float32),
        mesh=plsc.VectorSubcoreMesh(core_axis_name="c", subcore_axis_name="s"),
        scratch_types=(
            [pltpu.VMEM((32, 1024), jnp.float32)] * 2
            + [pltpu.VMEM((32, 128), jnp.float32)] * 2
            + [pltpu.VMEM((_CH, 256), jnp.float32)] * 2
            + [pltpu.SemaphoreType.DMA] * 4
        ),
        compiler_params=pltpu.CompilerParams(use_tc_tiling_on_sc=True,
                                             needs_layout_passes=False),
    )
    return f(rho)


# speedup vs baseline: 2.8094x; 1.1571x over previous
import jax
import jax.numpy as jnp
from jax import lax
from jax.experimental import pallas as pl
from jax.experimental.pallas import tpu as pltpu
from jax.experimental.pallas import tpu_sc as plsc

_CH = 16
_HALF = 128
_NCHUNK = _HALF // _CH


def _qpool_body(rho_hbm, out_hbm,
                rbuf0, rbuf1, cdbuf0, cdbuf1, obuf0, obuf1,
                semr0, semr1, semc0, semc1):
    cid = lax.axis_index("c")
    sid = lax.axis_index("s")
    lanes = lax.iota(jnp.int32, 16)

    ins = [(rbuf0, cdbuf0, semr0, semc0), (rbuf1, cdbuf1, semr1, semc1)]
    obufs = [obuf0, obuf1]

    def issue(k, p):
        rbuf, cdbuf, semr, semc = ins[p]
        i0 = 8 * cid + k
        r0 = 64 * i0
        pltpu.async_copy(rho_hbm.at[sid, pl.ds(r0, 32)], rbuf, semr)
        pltpu.async_copy(
            rho_hbm.at[sid, pl.ds(r0 + 32, 32),
                       pl.ds((r0 + 32) // 128 * 128, 128)],
            cdbuf, semc)

    def wait_in(p):
        rbuf, cdbuf, semr, semc = ins[p]
        pltpu.make_async_copy(rho_hbm.at[sid, pl.ds(0, 32)],
                              rbuf, semr).wait()
        pltpu.make_async_copy(rho_hbm.at[sid, pl.ds(0, 32), pl.ds(0, 128)],
                              cdbuf, semc).wait()

    def compute(k, p):
        rbuf, cdbuf, _, _ = ins[p]
        obuf = obufs[p]
        i0 = 8 * cid + k
        base16 = 16 * i0
        off = (64 * i0 + 32) % 128

        def row_body(t, carry2):
            tf = jnp.full((16,), t, jnp.int32)
            te = 2 * tf
            to = te + 1

            for lp in range(16):
                av = plsc.load_gather(rbuf, [te, 64 * lp + 2 * lanes])
                obuf[t, pl.ds(16 * lp, 16)] = av

            bv = plsc.load_gather(rbuf, [to, 64 * lanes + 2 * t + 1])
            plsc.addupdate_scatter(obuf, [tf, 16 * lanes + t], bv)

            cv = plsc.load_gather(cdbuf, [te, off + 2 * lanes])
            dv = plsc.load_gather(cdbuf, [to, jnp.full((16,), off,
                                                       jnp.int32) + 2 * t + 1])
            cd = cv + jnp.where(lanes == t, dv, jnp.float32(0))
            plsc.addupdate_scatter(obuf, [tf, base16 + lanes], cd)
            return carry2
        lax.fori_loop(0, _CH, row_body, 0)

        pltpu.sync_copy(obuf, out_hbm.at[sid, pl.ds(base16, _CH)])

    issue(0, 0)
    issue(1, 1)

    def pair_body(kk, carry):
        for p in range(2):
            k = 2 * kk + p
            wait_in(p)
            compute(k, p)

            @pl.when(kk < (_NCHUNK // 2) - 1)
            def _():
                issue(k + 2, p)
        return carry
    lax.fori_loop(0, _NCHUNK // 2, pair_body, 0)


def kernel(rho, mask_x, mask_y, new_x, new_y):
    b = rho.shape[0]

    f = pl.kernel(
        _qpool_body,
        out_type=jax.ShapeDtypeStruct((b, 256, 256), jnp.float32),
        mesh=plsc.VectorSubcoreMesh(core_axis_name="c", subcore_axis_name="s"),
        scratch_types=(
            [pltpu.VMEM((32, 1024), jnp.float32)] * 2
            + [pltpu.VMEM((32, 128), jnp.float32)] * 2
            + [pltpu.VMEM((_CH, 256), jnp.float32)] * 2
            + [pltpu.SemaphoreType.DMA] * 4
        ),
        compiler_params=pltpu.CompilerParams(use_tc_tiling_on_sc=True,
                                             needs_layout_passes=False),
    )
    return f(rho)
